# baseline (speedup 1.0000x reference)
import functools

import jax
import jax.numpy as jnp
from jax import lax
from jax.experimental import pallas as pl
from jax.experimental.pallas import tpu as pltpu
from jax.experimental.pallas import tpu_sc as plsc

_L = 16
_NC = 2
_NS = 16
_NW = _NC * _NS

_PARAMS = pltpu.CompilerParams(
    needs_layout_passes=False,
    use_tc_tiling_on_sc=True,
    disable_bounds_checks=True,
)


def _transpose_stage(table_t, edge128, *, H, V, R):
    VT = V // 128
    V_REM = V - VT * 128
    G = (H * V) // 4
    GPM = V // 4

    mesh = plsc.VectorSubcoreMesh(core_axis_name="c", subcore_axis_name="s")

    @functools.partial(
        pl.kernel,
        out_type=jax.ShapeDtypeStruct((G, 128), jnp.float32),
        mesh=mesh,
        compiler_params=_PARAMS,
        scratch_types=dict(
            ins=pltpu.VMEM((6, R, 128), jnp.float32),
            outs=pltpu.VMEM((6, 32, 128), jnp.float32),
            lsem0=pltpu.SemaphoreType.DMA,
            lsem1=pltpu.SemaphoreType.DMA,
            lsem2=pltpu.SemaphoreType.DMA,
            lsem3=pltpu.SemaphoreType.DMA,
            lsem4=pltpu.SemaphoreType.DMA,
            lsem5=pltpu.SemaphoreType.DMA,
            osem0=pltpu.SemaphoreType.DMA,
            osem1=pltpu.SemaphoreType.DMA,
            osem2=pltpu.SemaphoreType.DMA,
            osem3=pltpu.SemaphoreType.DMA,
            osem4=pltpu.SemaphoreType.DMA,
            osem5=pltpu.SemaphoreType.DMA,
        ),
    )
    def run(tab_hbm, edge_hbm, t128_hbm, *, ins, outs,
            lsem0, lsem1, lsem2, lsem3, lsem4, lsem5,
            osem0, osem1, osem2, osem3, osem4, osem5):
        wid = lax.axis_index("s") * _NC + lax.axis_index("c")

        base_cnt = VT // _NW
        extra = VT - base_cnt * _NW
        vt_cnt = jnp.where(wid < extra, base_cnt + 1, base_cnt)
        vt_lo = wid * base_cnt + jnp.minimum(wid, extra)
        n_jobs = H * vt_cnt

        iota = lax.iota(jnp.int32, _L)
        NBUF = 6
        lsems = (lsem0, lsem1, lsem2, lsem3, lsem4, lsem5)
        osems = (osem0, osem1, osem2, osem3, osem4, osem5)

        def job(k):
            return k // vt_cnt, vt_lo + lax.rem(k, vt_cnt)

        def fire_in(m, vt, s):
            pltpu.async_copy(
                tab_hbm.at[m, :, pl.ds(vt * 128, 128)], ins.at[s], lsems[s])

        def wait_in(s):
            pltpu.make_async_copy(
                tab_hbm.at[0, :, pl.ds(0, 128)], ins.at[s], lsems[s]).wait()

        def drain_out(s):
            pltpu.make_async_copy(
                outs.at[s], t128_hbm.at[pl.ds(0, 32)], osems[s]).wait()

        def transpose(n_g, s):
            in_r, out_r = ins.at[s], outs.at[s]
            row_lo = iota
            row_hi = iota + 16
            for g in range(n_g):
                for l in range(8):
                    rows = row_hi if l % 2 else row_lo
                    cols = jnp.broadcast_to(
                        jnp.int32(4 * g + l // 2), (_L,))
                    out_r[g, pl.ds(16 * l, 16)] = plsc.load_gather(
                        in_r, [rows, cols])

        def fire_out(m, vt, s):
            pltpu.async_copy(
                outs.at[s], t128_hbm.at[pl.ds(m * GPM + vt * 32, 32)],
                osems[s])

        for s in range(NBUF):
            m0, vt0 = job(s)
            fire_in(m0, vt0, s)

        @pl.loop(0, n_jobs, step=NBUF)
        def _(k):
            for s in range(NBUF):
                kk = k + s

                @pl.when(kk < n_jobs)
                def _():
                    m, vt = job(kk)

                    @pl.when(kk >= NBUF)
                    def _():
                        drain_out(s)

                    wait_in(s)
                    transpose(32, s)

                    @pl.when(kk + NBUF < n_jobs)
                    def _():
                        m2, vt2 = job(kk + NBUF)
                        fire_in(m2, vt2, s)

                    fire_out(m, vt, s)

        for s in range(NBUF):
            drain_out(s)

        if V_REM:
            n_e = V_REM // 4

            @pl.when(wid < H)
            def _():
                pltpu.sync_copy(edge_hbm.at[wid], outs.at[0, pl.ds(0, n_e)])
                pltpu.sync_copy(outs.at[0, pl.ds(0, n_e)],
                                t128_hbm.at[pl.ds(wid * GPM + VT * 32, n_e)])

    return run(table_t, edge128)


def _gather_stage(t128, coords_flat, w_splat, bias_splat, dummy,
                  *, B, H, V, R):
    BPW = B // _NW
    C = _L
    NCHUNK = BPW // C
    GPM = V // 4

    mesh = plsc.VectorSubcoreMesh(core_axis_name="c", subcore_axis_name="s")

    @functools.partial(
        pl.kernel,
        out_type=jax.ShapeDtypeStruct((B,), jnp.float32),
        mesh=mesh,
        compiler_params=_PARAMS,
        scratch_types=dict(
            coords_v=pltpu.VMEM((BPW * H,), jnp.int32),
            buf0=pltpu.VMEM((H, C, 128), jnp.float32),
            buf1=pltpu.VMEM((H, C, 128), jnp.float32),
            w_v=pltpu.VMEM((R, 128), jnp.float32),
            b_v=pltpu.VMEM((128,), jnp.float32),
            out_v=pltpu.VMEM((BPW,), jnp.float32),
            sem0=pltpu.SemaphoreType.DMA,
            sem1=pltpu.SemaphoreType.DMA,
        ),
    )
    def run(t128_hbm, coords_hbm, w_hbm, b_hbm, dummy_hbm, out_hbm, *,
            coords_v, buf0, buf1, w_v, b_v, out_v, sem0, sem1):
        wid = lax.axis_index("s") * _NC + lax.axis_index("c")
        base = wid * BPW

        pltpu.sync_copy(coords_hbm.at[pl.ds(base * H, BPW * H)], coords_v)
        pltpu.sync_copy(w_hbm, w_v)
        pltpu.sync_copy(b_hbm, b_v)

        iota = lax.iota(jnp.int32, _L)
        iota_h = iota * H

        buf_refs = (buf0, buf1)
        sems = (sem0, sem1)

        def coords16(c, m):
            return plsc.load_gather(coords_v, [iota_h + (c * C * H + m)])

        def fire(c, s):
            buf_r, sem = buf_refs[s], sems[s]

            @pl.loop(0, H)
            def _(m):
                cv = coords16(c, m)
                rows = m * GPM + lax.shift_right_logical(cv, 2)
                pltpu.async_copy(t128_hbm.at[rows], buf_r.at[m], sem)

        def drain(s):
            pltpu.make_async_copy(dummy_hbm, buf_refs[s], sems[s]).wait()

        def compute(c, s):
            buf_r = buf_refs[s]

            def load_rows(m, q32, r):
                return plsc.load_gather(
                    buf_r,
                    [jnp.full((_L,), m, jnp.int32), iota, q32 + r])

            q32_0 = lax.shift_left(jnp.bitwise_and(coords16(c, 0), 3), 5)
            acc = tuple(load_rows(0, q32_0, r) for r in range(R))

            def mbody(m, acc):
                q32 = lax.shift_left(jnp.bitwise_and(coords16(c, m), 3), 5)
                return tuple(acc[r] * load_rows(m, q32, r)
                             for r in range(R))

            acc = lax.fori_loop(1, H, mbody, acc)
            y = b_v[pl.ds(0, _L)]
            for r in range(R):
                y = y + w_v[r, pl.ds(0, _L)] * acc[r]
            out_v[pl.ds(c * C, _L)] = y

        fire(0, 0)

        @pl.loop(0, NCHUNK, step=2)
        def _(cc):
            fire(cc + 1, 1)
            drain(0)
            compute(cc, 0)

            @pl.when(cc + 2 < NCHUNK)
            def _():
                fire(cc + 2, 0)

            drain(1)
            compute(cc + 1, 1)

        pltpu.sync_copy(out_v, out_hbm.at[pl.ds(base, BPW)])

    return run(t128, coords_flat, w_splat, bias_splat, dummy)


def kernel(coords, factors, weights, bias):
    B, H = coords.shape
    _, V, R = factors.shape
    table_t = jnp.transpose(factors, (0, 2, 1))
    coords_flat = coords.astype(jnp.int32).reshape(-1)
    w_splat = jnp.broadcast_to(
        weights.astype(jnp.float32)[:, None], (R, 128))
    bias_splat = jnp.broadcast_to(bias.astype(jnp.float32), (128,))
    dummy = jnp.zeros((H, _L, 128), jnp.float32)
    VT = V // 128
    V_REM = V - VT * 128
    edge128 = factors[:, VT * 128:, :].reshape(H, V_REM // 4, 4 * R)
    t128 = _transpose_stage(table_t, edge128, H=H, V=V, R=R)
    return _gather_stage(t128, coords_flat, w_splat, bias_splat, dummy,
                         B=B, H=H, V=V, R=R)


# speedup vs baseline: 4.8849x; 4.8849x over previous
import functools

import jax
import jax.numpy as jnp
from jax import lax
from jax.experimental import pallas as pl
from jax.experimental.pallas import tpu as pltpu
from jax.experimental.pallas import tpu_sc as plsc

_L = 16
_NC = 2
_NS = 16
_NW = _NC * _NS

_PARAMS = pltpu.CompilerParams(
    needs_layout_passes=False,
    use_tc_tiling_on_sc=True,
    disable_bounds_checks=True,
)


def _transpose_stage(table_t, edge128, *, H, V, R):
    VT = V // 128
    V_REM = V - VT * 128
    G = (H * V) // 4
    GPM = V // 4

    mesh = plsc.VectorSubcoreMesh(core_axis_name="c", subcore_axis_name="s")

    @functools.partial(
        pl.kernel,
        out_type=jax.ShapeDtypeStruct((G, 128), jnp.float32),
        mesh=mesh,
        compiler_params=_PARAMS,
        scratch_types=dict(
            ins=pltpu.VMEM((6, R, 128), jnp.float32),
            outs=pltpu.VMEM((6, 32, 128), jnp.float32),
            lsem0=pltpu.SemaphoreType.DMA,
            lsem1=pltpu.SemaphoreType.DMA,
            lsem2=pltpu.SemaphoreType.DMA,
            lsem3=pltpu.SemaphoreType.DMA,
            lsem4=pltpu.SemaphoreType.DMA,
            lsem5=pltpu.SemaphoreType.DMA,
            osem0=pltpu.SemaphoreType.DMA,
            osem1=pltpu.SemaphoreType.DMA,
            osem2=pltpu.SemaphoreType.DMA,
            osem3=pltpu.SemaphoreType.DMA,
            osem4=pltpu.SemaphoreType.DMA,
            osem5=pltpu.SemaphoreType.DMA,
        ),
    )
    def run(tab_hbm, edge_hbm, t128_hbm, *, ins, outs,
            lsem0, lsem1, lsem2, lsem3, lsem4, lsem5,
            osem0, osem1, osem2, osem3, osem4, osem5):
        wid = lax.axis_index("s") * _NC + lax.axis_index("c")

        base_cnt = VT // _NW
        extra = VT - base_cnt * _NW
        vt_cnt = jnp.where(wid < extra, base_cnt + 1, base_cnt)
        vt_lo = wid * base_cnt + jnp.minimum(wid, extra)
        n_jobs = H * vt_cnt

        iota = lax.iota(jnp.int32, _L)
        NBUF = 6
        lsems = (lsem0, lsem1, lsem2, lsem3, lsem4, lsem5)
        osems = (osem0, osem1, osem2, osem3, osem4, osem5)

        def job(k):
            return k // vt_cnt, vt_lo + lax.rem(k, vt_cnt)

        def fire_in(m, vt, s):
            pltpu.async_copy(
                tab_hbm.at[m, :, pl.ds(vt * 128, 128)], ins.at[s], lsems[s])

        def wait_in(s):
            pltpu.make_async_copy(
                tab_hbm.at[0, :, pl.ds(0, 128)], ins.at[s], lsems[s]).wait()

        def drain_out(s):
            pltpu.make_async_copy(
                outs.at[s], t128_hbm.at[pl.ds(0, 32)], osems[s]).wait()

        def transpose(n_g, s):
            in_r, out_r = ins.at[s], outs.at[s]
            row_lo = iota
            row_hi = iota + 16

            @functools.partial(plsc.parallel_loop, 0, n_g, unroll=4)
            def _(g):
                for l in range(8):
                    rows = row_hi if l % 2 else row_lo
                    cols = jnp.broadcast_to(
                        (4 * g + l // 2).astype(jnp.int32), (_L,))
                    out_r[g, pl.ds(16 * l, 16)] = plsc.load_gather(
                        in_r, [rows, cols])

        def fire_out(m, vt, s):
            pltpu.async_copy(
                outs.at[s], t128_hbm.at[pl.ds(m * GPM + vt * 32, 32)],
                osems[s])

        for s in range(NBUF):
            m0, vt0 = job(s)
            fire_in(m0, vt0, s)

        @pl.loop(0, n_jobs, step=NBUF)
        def _(k):
            for s in range(NBUF):
                kk = k + s

                @pl.when(kk < n_jobs)
                def _():
                    m, vt = job(kk)

                    @pl.when(kk >= NBUF)
                    def _():
                        drain_out(s)

                    wait_in(s)
                    transpose(32, s)

                    @pl.when(kk + NBUF < n_jobs)
                    def _():
                        m2, vt2 = job(kk + NBUF)
                        fire_in(m2, vt2, s)

                    fire_out(m, vt, s)

        for s in range(NBUF):
            drain_out(s)

        if V_REM:
            n_e = V_REM // 4

            @pl.when(wid < H)
            def _():
                pltpu.sync_copy(edge_hbm.at[wid], outs.at[0, pl.ds(0, n_e)])
                pltpu.sync_copy(outs.at[0, pl.ds(0, n_e)],
                                t128_hbm.at[pl.ds(wid * GPM + VT * 32, n_e)])

    return run(table_t, edge128)


def _gather_stage(t128, coords_flat, w_splat, bias_splat, dummy,
                  *, B, H, V, R):
    BPW = B // _NW
    C = _L
    NCHUNK = BPW // C
    GPM = V // 4

    mesh = plsc.VectorSubcoreMesh(core_axis_name="c", subcore_axis_name="s")

    @functools.partial(
        pl.kernel,
        out_type=jax.ShapeDtypeStruct((B,), jnp.float32),
        mesh=mesh,
        compiler_params=_PARAMS,
        scratch_types=dict(
            coords_v=pltpu.VMEM((BPW * H,), jnp.int32),
            buf0=pltpu.VMEM((H, C, 128), jnp.float32),
            buf1=pltpu.VMEM((H, C, 128), jnp.float32),
            w_v=pltpu.VMEM((R, 128), jnp.float32),
            b_v=pltpu.VMEM((128,), jnp.float32),
            out_v=pltpu.VMEM((BPW,), jnp.float32),
            sem0=pltpu.SemaphoreType.DMA,
            sem1=pltpu.SemaphoreType.DMA,
        ),
    )
    def run(t128_hbm, coords_hbm, w_hbm, b_hbm, dummy_hbm, out_hbm, *,
            coords_v, buf0, buf1, w_v, b_v, out_v, sem0, sem1):
        wid = lax.axis_index("s") * _NC + lax.axis_index("c")
        base = wid * BPW

        pltpu.sync_copy(coords_hbm.at[pl.ds(base * H, BPW * H)], coords_v)
        pltpu.sync_copy(w_hbm, w_v)
        pltpu.sync_copy(b_hbm, b_v)

        iota = lax.iota(jnp.int32, _L)
        iota_h = iota * H

        buf_refs = (buf0, buf1)
        sems = (sem0, sem1)

        def coords16(c, m):
            return plsc.load_gather(coords_v, [iota_h + (c * C * H + m)])

        def fire(c, s):
            buf_r, sem = buf_refs[s], sems[s]

            @pl.loop(0, H)
            def _(m):
                cv = coords16(c, m)
                rows = m * GPM + lax.shift_right_logical(cv, 2)
                pltpu.async_copy(t128_hbm.at[rows], buf_r.at[m], sem)

        def drain(s):
            pltpu.make_async_copy(dummy_hbm, buf_refs[s], sems[s]).wait()

        def compute(c, s):
            buf_r = buf_refs[s]

            def load_rows(m, q32, r):
                return plsc.load_gather(
                    buf_r,
                    [jnp.full((_L,), m, jnp.int32), iota, q32 + r])

            q32_0 = lax.shift_left(jnp.bitwise_and(coords16(c, 0), 3), 5)
            acc = tuple(load_rows(0, q32_0, r) for r in range(R))

            def mbody(m, acc):
                q32 = lax.shift_left(jnp.bitwise_and(coords16(c, m), 3), 5)
                return tuple(acc[r] * load_rows(m, q32, r)
                             for r in range(R))

            acc = lax.fori_loop(1, H, mbody, acc)
            y = b_v[pl.ds(0, _L)]
            for r in range(R):
                y = y + w_v[r, pl.ds(0, _L)] * acc[r]
            out_v[pl.ds(c * C, _L)] = y

        fire(0, 0)

        @pl.loop(0, NCHUNK, step=2)
        def _(cc):
            fire(cc + 1, 1)
            drain(0)
            compute(cc, 0)

            @pl.when(cc + 2 < NCHUNK)
            def _():
                fire(cc + 2, 0)

            drain(1)
            compute(cc + 1, 1)

        pltpu.sync_copy(out_v, out_hbm.at[pl.ds(base, BPW)])

    return run(t128, coords_flat, w_splat, bias_splat, dummy)


def kernel(coords, factors, weights, bias):
    B, H = coords.shape
    _, V, R = factors.shape
    table_t = jnp.transpose(factors, (0, 2, 1))
    coords_flat = coords.astype(jnp.int32).reshape(-1)
    w_splat = jnp.broadcast_to(
        weights.astype(jnp.float32)[:, None], (R, 128))
    bias_splat = jnp.broadcast_to(bias.astype(jnp.float32), (128,))
    dummy = jnp.zeros((H, _L, 128), jnp.float32)
    VT = V // 128
    V_REM = V - VT * 128
    edge128 = factors[:, VT * 128:, :].reshape(H, V_REM // 4, 4 * R)
    t128 = _transpose_stage(table_t, edge128, H=H, V=V, R=R)
    return _gather_stage(t128, coords_flat, w_splat, bias_splat, dummy,
                         B=B, H=H, V=V, R=R)
